# baseline (speedup 1.0000x reference)
import functools

import jax
import jax.numpy as jnp
from jax import lax
from jax.experimental import pallas as pl
from jax.experimental.pallas import tpu as pltpu
from jax.experimental.pallas import tpu_sc as plsc

_NC = 2
_NS = 16
_NW = _NC * _NS
_L = 16
_CH = 128


def _center_loss_partials(y2, feat, centers, *, b_per_w, n_chunks, d):
    n_col = d // _L
    mesh = plsc.VectorSubcoreMesh(core_axis_name="c", subcore_axis_name="s")

    @functools.partial(
        pl.kernel,
        mesh=mesh,
        out_type=jax.ShapeDtypeStruct((_NW, _L), jnp.float32),
        scratch_types=[
            pltpu.VMEM((n_chunks, _CH), jnp.int32),
            pltpu.VMEM((b_per_w, d), jnp.float32),
            pltpu.VMEM((b_per_w, d), jnp.float32),
            pltpu.VMEM((_L,), jnp.float32),
            pltpu.SemaphoreType.DMA,
        ],
        compiler_params=pltpu.CompilerParams(use_tc_tiling_on_sc=False),
    )
    def k(y_hbm, feat_hbm, cent_hbm, out_hbm, idx_v, rows_v, feat_v, acc_v,
          sem):
        wid = lax.axis_index("s") * _NC + lax.axis_index("c")
        base = wid * b_per_w

        pltpu.sync_copy(y_hbm.at[pl.ds(wid * n_chunks, n_chunks)], idx_v)
        gathers = []
        for j in range(n_chunks):
            gathers.append(
                pltpu.async_copy(
                    cent_hbm.at[idx_v.at[j]],
                    rows_v.at[pl.ds(j * _CH, _CH)],
                    sem,
                ))
        pltpu.sync_copy(feat_hbm.at[pl.ds(base, b_per_w)], feat_v)
        for g in gathers:
            g.wait()

        def body(i, accs):
            out = []
            for j in range(n_col):
                f = feat_v[i, pl.ds(j * _L, _L)]
                c = rows_v[i, pl.ds(j * _L, _L)]
                diff = f - c
                out.append(accs[j] + diff * diff)
            return tuple(out)

        zero = jnp.zeros((_L,), jnp.float32)
        accs = lax.fori_loop(0, b_per_w, body, (zero,) * n_col)
        total = accs[0]
        for j in range(1, n_col):
            total = total + accs[j]
        acc_v[...] = total * 0.5
        pltpu.sync_copy(acc_v, out_hbm.at[wid])

    return k(y2, feat, centers)


def kernel(y, feat, centers):
    b, d = feat.shape
    b_per_w = b // _NW
    n_chunks = b_per_w // _CH
    y2 = y.reshape(_NW * n_chunks, _CH)
    partials = _center_loss_partials(
        y2, feat, centers, b_per_w=b_per_w, n_chunks=n_chunks, d=d)
    return jnp.sum(partials)


# speedup vs baseline: 1.2258x; 1.2258x over previous
import functools

import jax
import jax.numpy as jnp
from jax import lax
from jax.experimental import pallas as pl
from jax.experimental.pallas import tpu as pltpu
from jax.experimental.pallas import tpu_sc as plsc

_NC = 2
_NS = 16
_NW = _NC * _NS
_L = 16
_CHUNK = 32


def _center_loss_partials(y, feat, centers, *, b_per_w, d):
    n_col = d // _L
    n_chunks = b_per_w // _CHUNK
    n_pairs = n_chunks // 2
    mesh = plsc.VectorSubcoreMesh(core_axis_name="c", subcore_axis_name="s")

    @functools.partial(
        pl.kernel,
        mesh=mesh,
        out_type=jax.ShapeDtypeStruct((_NW, _L), jnp.float32),
        scratch_types=[
            pltpu.VMEM((b_per_w,), jnp.int32),
            pltpu.VMEM((2, _CHUNK, d), jnp.float32),
            pltpu.VMEM((b_per_w, d), jnp.float32),
            pltpu.VMEM((_L,), jnp.float32),
            pltpu.SemaphoreType.DMA,
            pltpu.SemaphoreType.DMA,
            pltpu.SemaphoreType.DMA,
        ],
    )
    def k(y_hbm, feat_hbm, cent_hbm, out_hbm, yv, rows_v, feat_v,
          acc_v, sem_a, sem_b, sem_f):
        wid = lax.axis_index("s") * _NC + lax.axis_index("c")
        base = wid * b_per_w

        pltpu.sync_copy(y_hbm.at[pl.ds(base, b_per_w)], yv)
        feat_cp = pltpu.async_copy(feat_hbm.at[pl.ds(base, b_per_w)], feat_v,
                                   sem_f)

        def issue(chunk, buf, sem):
            off = chunk * _CHUNK
            for g in range(_CHUNK // _L):
                idx_vec = yv[pl.ds(off + g * _L, _L)]
                for lane in range(_L):
                    j = g * _L + lane
                    pltpu.async_copy(cent_hbm.at[pl.ds(idx_vec[lane], 1)],
                                     rows_v.at[buf, pl.ds(j, 1)], sem)

        def drain(buf, sem):
            pltpu.make_async_copy(cent_hbm.at[pl.ds(0, _CHUNK)],
                                  rows_v.at[buf], sem).wait()

        def accum(chunk, buf, accs):
            off = chunk * _CHUNK
            for j in range(_CHUNK):
                for col in range(n_col):
                    f = feat_v[off + j, pl.ds(col * _L, _L)]
                    c = rows_v[buf, j, pl.ds(col * _L, _L)]
                    diff = f - c
                    accs[col] = accs[col] + diff * diff
            return accs

        issue(0, 0, sem_a)
        feat_cp.wait()

        def body(p, accs):
            accs = list(accs)
            a = p * 2
            issue(a + 1, 1, sem_b)
            drain(0, sem_a)
            accs = accum(a, 0, accs)

            @pl.when(p < n_pairs - 1)
            def _():
                issue(a + 2, 0, sem_a)

            drain(1, sem_b)
            accs = accum(a + 1, 1, accs)
            return tuple(accs)

        zero = jnp.zeros((_L,), jnp.float32)
        accs = lax.fori_loop(0, n_pairs, body, (zero,) * n_col)
        total = accs[0]
        for j in range(1, n_col):
            total = total + accs[j]
        acc_v[...] = total * 0.5
        pltpu.sync_copy(acc_v, out_hbm.at[wid])

    return k(y, feat, centers)


def kernel(y, feat, centers):
    b, d = feat.shape
    b_per_w = b // _NW
    partials = _center_loss_partials(y, feat, centers, b_per_w=b_per_w, d=d)
    return jnp.sum(partials)


# speedup vs baseline: 1.9654x; 1.6033x over previous
import functools

import jax
import jax.numpy as jnp
from jax import lax
from jax.experimental import pallas as pl
from jax.experimental.pallas import tpu as pltpu
from jax.experimental.pallas import tpu_sc as plsc

_NC = 2
_NS = 16
_NW = _NC * _NS
_L = 16
_CHUNK = 4096


def _center_loss_partials(y, feat_t, centers_t, *, b, v, d):
    f_per_w = d // _NW
    n_chunks = b // _CHUNK
    mesh = plsc.VectorSubcoreMesh(core_axis_name="c", subcore_axis_name="s")

    @functools.partial(
        pl.kernel,
        mesh=mesh,
        out_type=jax.ShapeDtypeStruct((_NW, _L), jnp.float32),
        scratch_types=[
            pltpu.VMEM((v,), jnp.float32),
            pltpu.VMEM((2, _CHUNK), jnp.int32),
            pltpu.VMEM((2, _CHUNK), jnp.float32),
            pltpu.VMEM((_L,), jnp.float32),
            pltpu.SemaphoreType.DMA,
            pltpu.SemaphoreType.DMA,
            pltpu.SemaphoreType.DMA,
        ],
        compiler_params=pltpu.CompilerParams(needs_layout_passes=False),
    )
    def k(y_hbm, feat_hbm, cent_hbm, out_hbm, row_v, y_v, f_v, acc_v,
          sem_row, sem_a, sem_b):
        wid = lax.axis_index("s") * _NC + lax.axis_index("c")

        def stage(feature, chunk, buf, sem):
            off = chunk * _CHUNK
            pltpu.async_copy(y_hbm.at[pl.ds(off, _CHUNK)], y_v.at[buf], sem)
            pltpu.async_copy(feat_hbm.at[feature, pl.ds(off, _CHUNK)],
                             f_v.at[buf], sem)

        def drain(buf, sem):
            pltpu.make_async_copy(y_hbm.at[pl.ds(0, _CHUNK)], y_v.at[buf],
                                  sem).wait()
            pltpu.make_async_copy(
                feat_hbm.at[0, pl.ds(0, _CHUNK)], f_v.at[buf], sem).wait()

        def accum(buf, accs):
            def body(i, accs):
                out = []
                idx = y_v[buf, pl.ds(i * _L, _L)]
                fv = f_v[buf, pl.ds(i * _L, _L)]
                cv = plsc.load_gather(row_v, [idx])
                diff = fv - cv
                out.append(accs[0] + diff * diff)
                return tuple(out)

            return list(lax.fori_loop(0, _CHUNK // _L, body, tuple(accs)))

        accs = [jnp.zeros((_L,), jnp.float32)]
        for t in range(f_per_w):
            feature = wid * f_per_w + t
            pltpu.async_copy(cent_hbm.at[feature], row_v, sem_row)
            stage(feature, 0, 0, sem_a)
            pltpu.make_async_copy(cent_hbm.at[0], row_v, sem_row).wait()
            for c in range(n_chunks):
                buf = c % 2
                nbuf = 1 - buf
                if c + 1 < n_chunks:
                    stage(feature, c + 1, nbuf, sem_b if nbuf else sem_a)
                drain(buf, sem_b if buf else sem_a)
                accs = accum(buf, accs)

        acc_v[...] = accs[0] * 0.5
        pltpu.sync_copy(acc_v, out_hbm.at[wid])

    return k(y, feat_t, centers_t)


def kernel(y, feat, centers):
    b, d = feat.shape
    v = centers.shape[0]
    partials = _center_loss_partials(y, feat.T, centers.T, b=b, v=v, d=d)
    return jnp.sum(partials)


# speedup vs baseline: 2.3805x; 1.2112x over previous
import functools

import jax
import jax.numpy as jnp
from jax import lax
from jax.experimental import pallas as pl
from jax.experimental.pallas import tpu as pltpu
from jax.experimental.pallas import tpu_sc as plsc

_NC = 2
_NS = 16
_NW = _NC * _NS
_L = 16
_CHUNK = 4096
_UNROLL = 4


def _center_loss_partials(y, feat_t, centers_t, *, b, v, d):
    f_per_w = d // _NW
    n_chunks = b // _CHUNK
    mesh = plsc.VectorSubcoreMesh(core_axis_name="c", subcore_axis_name="s")

    @functools.partial(
        pl.kernel,
        mesh=mesh,
        out_type=jax.ShapeDtypeStruct((_NW, _L), jnp.float32),
        scratch_types=[
            pltpu.VMEM((v,), jnp.float32),
            pltpu.VMEM((b,), jnp.int32),
            pltpu.VMEM((2, _CHUNK), jnp.float32),
            pltpu.VMEM((_L,), jnp.float32),
            pltpu.SemaphoreType.DMA,
            pltpu.SemaphoreType.DMA,
            pltpu.SemaphoreType.DMA,
        ],
        compiler_params=pltpu.CompilerParams(needs_layout_passes=False),
    )
    def k(y_hbm, feat_hbm, cent_hbm, out_hbm, row_v, y_v, f_v, acc_v,
          sem_row, sem_a, sem_b):
        wid = lax.axis_index("s") * _NC + lax.axis_index("c")

        pltpu.async_copy(cent_hbm.at[wid * f_per_w], row_v, sem_row)
        pltpu.sync_copy(y_hbm, y_v)

        def stage(feature, chunk, buf, sem):
            pltpu.async_copy(feat_hbm.at[feature, pl.ds(chunk * _CHUNK,
                                                        _CHUNK)],
                             f_v.at[buf], sem)

        def drain(buf, sem):
            pltpu.make_async_copy(
                feat_hbm.at[0, pl.ds(0, _CHUNK)], f_v.at[buf], sem).wait()

        def accum(chunk, buf, accs):
            base = chunk * _CHUNK

            def body(i, accs):
                off = base + i * (_L * _UNROLL)
                loc = i * (_L * _UNROLL)
                out = []
                for u in range(_UNROLL):
                    idx = y_v[pl.ds(off + u * _L, _L)]
                    fv = f_v[buf, pl.ds(loc + u * _L, _L)]
                    cv = plsc.load_gather(row_v, [idx])
                    diff = fv - cv
                    out.append(accs[u] + diff * diff)
                return tuple(out)

            return list(
                lax.fori_loop(0, _CHUNK // (_L * _UNROLL), body, tuple(accs)))

        accs = [jnp.zeros((_L,), jnp.float32)] * _UNROLL
        for t in range(f_per_w):
            feature = wid * f_per_w + t
            stage(feature, 0, 0, sem_a)
            pltpu.make_async_copy(cent_hbm.at[0], row_v, sem_row).wait()
            for c in range(n_chunks):
                buf = c % 2
                nbuf = 1 - buf
                if c + 1 < n_chunks:
                    stage(feature, c + 1, nbuf, sem_b if nbuf else sem_a)
                drain(buf, sem_b if buf else sem_a)
                accs = accum(c, buf, accs)
            if t + 1 < f_per_w:
                pltpu.async_copy(cent_hbm.at[feature + 1], row_v, sem_row)

        total = accs[0]
        for u in range(1, _UNROLL):
            total = total + accs[u]
        acc_v[...] = total * 0.5
        pltpu.sync_copy(acc_v, out_hbm.at[wid])

    return k(y, feat_t, centers_t)


def kernel(y, feat, centers):
    b, d = feat.shape
    v = centers.shape[0]
    partials = _center_loss_partials(y, feat.T, centers.T, b=b, v=v, d=d)
    return jnp.sum(partials)


# speedup vs baseline: 2.3904x; 1.0042x over previous
import functools

import jax
import jax.numpy as jnp
from jax import lax
from jax.experimental import pallas as pl
from jax.experimental.pallas import tpu as pltpu
from jax.experimental.pallas import tpu_sc as plsc

_NC = 2
_NS = 16
_NW = _NC * _NS
_L = 16
_CHUNK = 4096
_UNROLL = 4


def _center_loss_partials(y, feat_t, centers_t, *, b, v, d):
    f_per_w = d // _NW
    n_chunks = b // _CHUNK
    mesh = plsc.VectorSubcoreMesh(core_axis_name="c", subcore_axis_name="s")

    @functools.partial(
        pl.kernel,
        mesh=mesh,
        out_type=jax.ShapeDtypeStruct((_NW, _L), jnp.float32),
        scratch_types=[
            pltpu.VMEM((v,), jnp.float32),
            pltpu.VMEM((b,), jnp.int32),
            pltpu.VMEM((2, _CHUNK), jnp.float32),
            pltpu.VMEM((_L,), jnp.float32),
            pltpu.SemaphoreType.DMA,
            pltpu.SemaphoreType.DMA,
            pltpu.SemaphoreType.DMA,
        ],
        compiler_params=pltpu.CompilerParams(needs_layout_passes=False),
    )
    def k(y_hbm, feat_hbm, cent_hbm, out_hbm, row_v, y_v, f_v, acc_v,
          sem_row, sem_a, sem_b):
        wid = lax.axis_index("s") * _NC + lax.axis_index("c")

        pltpu.async_copy(cent_hbm.at[wid * f_per_w], row_v, sem_row)
        pltpu.sync_copy(y_hbm, y_v)

        def stage(feature, chunk, buf, sem):
            pltpu.async_copy(feat_hbm.at[feature, pl.ds(chunk * _CHUNK,
                                                        _CHUNK)],
                             f_v.at[buf], sem)

        def drain(buf, sem):
            pltpu.make_async_copy(
                feat_hbm.at[0, pl.ds(0, _CHUNK)], f_v.at[buf], sem).wait()

        def accum(chunk, buf, accs):
            base = chunk * _CHUNK

            @plsc.parallel_loop(0, _CHUNK // _L, _UNROLL, unroll=2,
                                carry=tuple(accs))
            def body(i, accs):
                out = []
                for u in range(_UNROLL):
                    idx = y_v[pl.ds(base + (i + u) * _L, _L)]
                    fv = f_v[buf, pl.ds((i + u) * _L, _L)]
                    cv = plsc.load_gather(row_v, [idx])
                    diff = fv - cv
                    out.append(accs[u] + diff * diff)
                return tuple(out)

            return list(body)

        accs = [jnp.zeros((_L,), jnp.float32)] * _UNROLL
        for t in range(f_per_w):
            feature = wid * f_per_w + t
            stage(feature, 0, 0, sem_a)
            pltpu.make_async_copy(cent_hbm.at[0], row_v, sem_row).wait()
            for c in range(n_chunks):
                buf = c % 2
                nbuf = 1 - buf
                if c + 1 < n_chunks:
                    stage(feature, c + 1, nbuf, sem_b if nbuf else sem_a)
                drain(buf, sem_b if buf else sem_a)
                accs = accum(c, buf, accs)
            if t + 1 < f_per_w:
                pltpu.async_copy(cent_hbm.at[feature + 1], row_v, sem_row)

        total = accs[0]
        for u in range(1, _UNROLL):
            total = total + accs[u]
        acc_v[...] = total * 0.5
        pltpu.sync_copy(acc_v, out_hbm.at[wid])

    return k(y, feat_t, centers_t)


def kernel(y, feat, centers):
    b, d = feat.shape
    v = centers.shape[0]
    partials = _center_loss_partials(y, feat.T, centers.T, b=b, v=v, d=d)
    return jnp.sum(partials)
